# baseline (speedup 1.0000x reference)
import functools

import jax
import jax.numpy as jnp
from jax.experimental import pallas as pl
from jax.experimental.pallas import tpu as pltpu

TILE_N = 256


def _body(data_ref, w_ref, bias_ref, lr_ref, li_ref, real_ref, imag_ref,
          a_ref, b_ref, ab_ref, *, num_k):
    i = pl.program_id(0)
    k = pl.program_id(1)

    @pl.when(jnp.logical_and(i == 0, k == 0))
    def _init_ab():
        xr = data_ref[0].astype(jnp.bfloat16)
        xi = data_ref[1].astype(jnp.bfloat16)
        for kk in range(num_k):
            w = w_ref[kk].astype(jnp.bfloat16)
            a = jnp.dot(xr, w, preferred_element_type=jnp.float32)
            b = jnp.dot(xi, w, preferred_element_type=jnp.float32)
            a_ref[kk] = a.astype(jnp.bfloat16)
            b_ref[kk] = b.astype(jnp.bfloat16)
            ab_ref[kk] = (a + b).astype(jnp.bfloat16)

    lr = lr_ref[0].astype(jnp.bfloat16)
    li = li_ref[0].astype(jnp.bfloat16)
    lsum = lr + li
    t1 = jnp.dot(lr, a_ref[k], preferred_element_type=jnp.float32)
    t2 = jnp.dot(li, b_ref[k], preferred_element_type=jnp.float32)
    t3 = jnp.dot(lsum, ab_ref[k], preferred_element_type=jnp.float32)
    t_real = t1 - t2
    t_imag = t3 - t1 - t2

    @pl.when(k == 0)
    def _first():
        real_ref[...] = t_real + bias_ref[...]
        imag_ref[...] = t_imag + bias_ref[...]

    @pl.when(k != 0)
    def _acc():
        real_ref[...] += t_real
        imag_ref[...] += t_imag


def kernel(data, L_norm_real, L_norm_imag, weight, bias):
    num_k, n, _ = L_norm_real.shape
    f_in = data.shape[2]
    f_out = weight.shape[2]
    num_tiles = n // TILE_N

    grid = (num_tiles, num_k)
    out_shape = (
        jax.ShapeDtypeStruct((n, f_out), jnp.float32),
        jax.ShapeDtypeStruct((n, f_out), jnp.float32),
    )
    real, imag = pl.pallas_call(
        functools.partial(_body, num_k=num_k),
        grid=grid,
        in_specs=[
            pl.BlockSpec((2, n, f_in), lambda i, k: (0, 0, 0)),
            pl.BlockSpec((num_k, f_in, f_out), lambda i, k: (0, 0, 0)),
            pl.BlockSpec((1, f_out), lambda i, k: (0, 0)),
            pl.BlockSpec((1, TILE_N, n), lambda i, k: (k, i, 0)),
            pl.BlockSpec((1, TILE_N, n), lambda i, k: (k, i, 0)),
        ],
        out_specs=[
            pl.BlockSpec((TILE_N, f_out), lambda i, k: (i, 0)),
            pl.BlockSpec((TILE_N, f_out), lambda i, k: (i, 0)),
        ],
        out_shape=out_shape,
        scratch_shapes=[
            pltpu.VMEM((num_k, n, f_out), jnp.bfloat16),
            pltpu.VMEM((num_k, n, f_out), jnp.bfloat16),
            pltpu.VMEM((num_k, n, f_out), jnp.bfloat16),
        ],
    )(data, weight, bias, L_norm_real, L_norm_imag)
    return (real, imag)


# speedup vs baseline: 1.0993x; 1.0993x over previous
import functools

import jax
import jax.numpy as jnp
from jax.experimental import pallas as pl
from jax.experimental.pallas import tpu as pltpu

TILE_N = 128


def _body(data_ref, w_ref, bias_ref, lr_ref, li_ref, real_ref, imag_ref,
          a_ref, b_ref, ab_ref, *, num_k):
    i = pl.program_id(0)

    @pl.when(i == 0)
    def _init_ab():
        xr = data_ref[0].astype(jnp.bfloat16)
        xi = data_ref[1].astype(jnp.bfloat16)
        for kk in range(num_k):
            w = w_ref[kk].astype(jnp.bfloat16)
            a = jnp.dot(xr, w, preferred_element_type=jnp.float32)
            b = jnp.dot(xi, w, preferred_element_type=jnp.float32)
            a_ref[kk] = a.astype(jnp.bfloat16)
            b_ref[kk] = b.astype(jnp.bfloat16)
            ab_ref[kk] = (a + b).astype(jnp.bfloat16)

    real_acc = bias_ref[...].astype(jnp.float32)
    imag_acc = bias_ref[...].astype(jnp.float32)
    for kk in range(num_k):
        lr = lr_ref[kk].astype(jnp.bfloat16)
        li = li_ref[kk].astype(jnp.bfloat16)
        lsum = lr + li
        t1 = jnp.dot(lr, a_ref[kk], preferred_element_type=jnp.float32)
        t2 = jnp.dot(li, b_ref[kk], preferred_element_type=jnp.float32)
        t3 = jnp.dot(lsum, ab_ref[kk], preferred_element_type=jnp.float32)
        real_acc = real_acc + (t1 - t2)
        imag_acc = imag_acc + (t3 - t1 - t2)
    real_ref[...] = real_acc
    imag_ref[...] = imag_acc


def kernel(data, L_norm_real, L_norm_imag, weight, bias):
    num_k, n, _ = L_norm_real.shape
    f_in = data.shape[2]
    f_out = weight.shape[2]
    num_tiles = n // TILE_N

    grid = (num_tiles,)
    out_shape = (
        jax.ShapeDtypeStruct((n, f_out), jnp.float32),
        jax.ShapeDtypeStruct((n, f_out), jnp.float32),
    )
    real, imag = pl.pallas_call(
        functools.partial(_body, num_k=num_k),
        grid=grid,
        in_specs=[
            pl.BlockSpec((2, n, f_in), lambda i: (0, 0, 0)),
            pl.BlockSpec((num_k, f_in, f_out), lambda i: (0, 0, 0)),
            pl.BlockSpec((1, f_out), lambda i: (0, 0)),
            pl.BlockSpec((num_k, TILE_N, n), lambda i: (0, i, 0)),
            pl.BlockSpec((num_k, TILE_N, n), lambda i: (0, i, 0)),
        ],
        out_specs=[
            pl.BlockSpec((TILE_N, f_out), lambda i: (i, 0)),
            pl.BlockSpec((TILE_N, f_out), lambda i: (i, 0)),
        ],
        out_shape=out_shape,
        scratch_shapes=[
            pltpu.VMEM((num_k, n, f_out), jnp.bfloat16),
            pltpu.VMEM((num_k, n, f_out), jnp.bfloat16),
            pltpu.VMEM((num_k, n, f_out), jnp.bfloat16),
        ],
    )(data, weight, bias, L_norm_real, L_norm_imag)
    return (real, imag)


# speedup vs baseline: 1.1897x; 1.0822x over previous
import functools

import jax
import jax.numpy as jnp
from jax.experimental import pallas as pl
from jax.experimental.pallas import tpu as pltpu

TILE_N = 128
BUFFERS = 4


def _outer(data_ref, w_ref, bias_ref, lr_hbm, li_hbm, real_hbm, imag_hbm,
           a_ref, b_ref, ab_ref, *, num_k, n, f_out, num_tiles):
    xr = data_ref[0].astype(jnp.bfloat16)
    xi = data_ref[1].astype(jnp.bfloat16)
    for kk in range(num_k):
        w = w_ref[kk].astype(jnp.bfloat16)
        a = jnp.dot(xr, w, preferred_element_type=jnp.float32)
        b = jnp.dot(xi, w, preferred_element_type=jnp.float32)
        a_ref[kk] = a.astype(jnp.bfloat16)
        b_ref[kk] = b.astype(jnp.bfloat16)
        ab_ref[kk] = (a + b).astype(jnp.bfloat16)

    def inner(lr_ref, li_ref, real_ref, imag_ref):
        real_acc = bias_ref[...].astype(jnp.float32)
        imag_acc = bias_ref[...].astype(jnp.float32)
        for kk in range(num_k):
            lr = lr_ref[kk].astype(jnp.bfloat16)
            li = li_ref[kk].astype(jnp.bfloat16)
            lsum = lr + li
            t1 = jnp.dot(lr, a_ref[kk], preferred_element_type=jnp.float32)
            t2 = jnp.dot(li, b_ref[kk], preferred_element_type=jnp.float32)
            t3 = jnp.dot(lsum, ab_ref[kk], preferred_element_type=jnp.float32)
            real_acc = real_acc + (t1 - t2)
            imag_acc = imag_acc + (t3 - t1 - t2)
        real_ref[...] = real_acc
        imag_ref[...] = imag_acc

    pipeline = pltpu.emit_pipeline(
        inner,
        grid=(num_tiles,),
        in_specs=[
            pl.BlockSpec((num_k, TILE_N, n), lambda i: (0, i, 0),
                         pipeline_mode=pl.Buffered(buffer_count=BUFFERS)),
            pl.BlockSpec((num_k, TILE_N, n), lambda i: (0, i, 0),
                         pipeline_mode=pl.Buffered(buffer_count=BUFFERS)),
        ],
        out_specs=[
            pl.BlockSpec((TILE_N, f_out), lambda i: (i, 0)),
            pl.BlockSpec((TILE_N, f_out), lambda i: (i, 0)),
        ],
    )
    pipeline(lr_hbm, li_hbm, real_hbm, imag_hbm)


def kernel(data, L_norm_real, L_norm_imag, weight, bias):
    num_k, n, _ = L_norm_real.shape
    f_in = data.shape[2]
    f_out = weight.shape[2]
    num_tiles = n // TILE_N

    out_shape = (
        jax.ShapeDtypeStruct((n, f_out), jnp.float32),
        jax.ShapeDtypeStruct((n, f_out), jnp.float32),
    )
    real, imag = pl.pallas_call(
        functools.partial(_outer, num_k=num_k, n=n, f_out=f_out,
                          num_tiles=num_tiles),
        grid=(1,),
        in_specs=[
            pl.BlockSpec((2, n, f_in), lambda i: (0, 0, 0)),
            pl.BlockSpec((num_k, f_in, f_out), lambda i: (0, 0, 0)),
            pl.BlockSpec((1, f_out), lambda i: (0, 0)),
            pl.BlockSpec(memory_space=pl.ANY),
            pl.BlockSpec(memory_space=pl.ANY),
        ],
        out_specs=[
            pl.BlockSpec(memory_space=pl.ANY),
            pl.BlockSpec(memory_space=pl.ANY),
        ],
        out_shape=out_shape,
        scratch_shapes=[
            pltpu.VMEM((num_k, n, f_out), jnp.bfloat16),
            pltpu.VMEM((num_k, n, f_out), jnp.bfloat16),
            pltpu.VMEM((num_k, n, f_out), jnp.bfloat16),
        ],
    )(data, weight, bias, L_norm_real, L_norm_imag)
    return (real, imag)
